# baseline (speedup 1.0000x reference)
import jax
import jax.numpy as jnp
import numpy as np
from jax.experimental import pallas as pl
from jax.experimental.pallas import tpu as pltpu

B, H, W = 16, 512, 512
RS = 12
NC = 80


def _conv2d(x, w, b, s):
    y = jax.lax.conv_general_dilated(x, w, (s, s), ((1, 1), (1, 1)),
                                     dimension_numbers=('NCHW', 'OIHW', 'NCHW'))
    return y + b[None, :, None, None]


def _extract_roi_one(image, boxes):
    Hs, Ws = image.shape[1], image.shape[2]
    y0, x0, y1, x1 = boxes[:, 0], boxes[:, 1], boxes[:, 2], boxes[:, 3]
    bh = jnp.maximum(y1 - y0, 1.0)
    bw = jnp.maximum(x1 - x0, 1.0)
    k = (jnp.arange(RS, dtype=jnp.float32) + 0.5) / RS
    sy = y0[:, None] + k[None, :] * bh[:, None] - 0.5
    sx = x0[:, None] + k[None, :] * bw[:, None] - 0.5
    YY = sy[:, :, None]
    XX = sx[:, None, :]
    yf = jnp.floor(YY); xf = jnp.floor(XX)
    wy = YY - yf; wx = XX - xf
    y0i = jnp.clip(yf, 0, Hs - 1).astype(jnp.int32)
    y1i = jnp.clip(yf + 1, 0, Hs - 1).astype(jnp.int32)
    x0i = jnp.clip(xf, 0, Ws - 1).astype(jnp.int32)
    x1i = jnp.clip(xf + 1, 0, Ws - 1).astype(jnp.int32)
    g = lambda yi, xi: image[:, yi, xi]
    v = (g(y0i, x0i) * (1 - wy) * (1 - wx) + g(y0i, x1i) * (1 - wy) * wx
         + g(y1i, x0i) * wy * (1 - wx) + g(y1i, x1i) * wy * wx)
    return jnp.transpose(v, (1, 0, 2, 3))


def _pool_proj_kernel(scores_ref, lat_ref, ow_ref, ob_ref, out_ref):
    s = scores_ref[0]
    lat = lat_ref[0]
    h = jnp.dot(s, lat, preferred_element_type=jnp.float32)
    out_ref[0] = (jnp.dot(h, ow_ref[...], preferred_element_type=jnp.float32)
                  + ob_ref[...])


def _pool_proj(scores, lat, ow, ob):
    Bq, N, D = lat.shape
    out = pl.pallas_call(
        _pool_proj_kernel,
        grid=(Bq,),
        in_specs=[
            pl.BlockSpec((1, 1, N), lambda b: (b, 0, 0)),
            pl.BlockSpec((1, N, D), lambda b: (b, 0, 0)),
            pl.BlockSpec((D, NC), lambda b: (0, 0)),
            pl.BlockSpec((1, NC), lambda b: (0, 0)),
        ],
        out_specs=pl.BlockSpec((1, 1, NC), lambda b: (b, 0, 0)),
        out_shape=jax.ShapeDtypeStruct((Bq, 1, NC), jnp.float32),
        compiler_params=pltpu.CompilerParams(
            dimension_semantics=("arbitrary",),
        ),
    )(scores.reshape(Bq, 1, N), lat, ow, ob.reshape(1, NC))
    return out.reshape(Bq, NC)


def kernel(images, gw1, gb1, gw2, gb2, gw3, gb3, gw4, gb4,
           rw1, rb1, rw2, rb2, ow, ob):
    Bq, _, Hi, Wi = images.shape
    g = _conv2d(_conv2d(_conv2d(_conv2d(images, gw1, gb1, 2), gw2, gb2, 2),
                        gw3, gb3, 2), gw4, gb4, 2)
    Hg, Wg = g.shape[2], g.shape[3]
    rr, cc = jnp.meshgrid(jnp.arange(Hg), jnp.arange(Wg), indexing='ij')
    cells = jnp.stack([rr, cc]).astype(images.dtype)[None]
    cr = jnp.array([Hi / Hg, Wi / Wg], images.dtype)[None, :, None, None]
    sr = jnp.array([Hi, Wi], images.dtype)[None, :, None, None]
    centers = cr * (jax.nn.sigmoid(g[:, 0:2]) + cells)
    sizes = sr * jax.nn.sigmoid(g[:, 2:4])
    bb = jnp.concatenate([centers - sizes / 2, centers + sizes / 2], axis=1)
    bb = jnp.stack([jnp.maximum(bb[:, 0], 0.0),
                    jnp.maximum(bb[:, 1], 0.0),
                    jnp.minimum(bb[:, 2], Hi - 1.0),
                    jnp.minimum(bb[:, 3], Wi - 1.0)], axis=1)
    boxes = jnp.floor(bb)
    boxes = boxes.transpose(0, 2, 3, 1).reshape(Bq, Hg * Wg, 4)
    scores = jax.nn.softmax(g[:, 4].reshape(Bq, -1), axis=-1)
    regions = jax.vmap(_extract_roi_one)(images, boxes)
    regions = regions.reshape(Bq * Hg * Wg, 3, RS, RS)
    lat = _conv2d(_conv2d(regions, rw1, rb1, 2), rw2, rb2, 2)
    lat = lat.reshape(Bq, Hg * Wg, -1)
    logits = _pool_proj(scores, lat, ow, ob)
    return logits, scores.reshape(Bq, 1, Hg, Wg)


# speedup vs baseline: 1.8484x; 1.8484x over previous
import jax
import jax.numpy as jnp
import numpy as np
from jax.experimental import pallas as pl
from jax.experimental.pallas import tpu as pltpu

B, H, W = 16, 512, 512
RS = 12
NC = 80


def _conv2d(x, w, b, s):
    y = jax.lax.conv_general_dilated(x, w, (s, s), ((1, 1), (1, 1)),
                                     dimension_numbers=('NCHW', 'OIHW', 'NCHW'))
    return y + b[None, :, None, None]


def _extract_roi_one(image, boxes):
    Hs, Ws = image.shape[1], image.shape[2]
    y0, x0, y1, x1 = boxes[:, 0], boxes[:, 1], boxes[:, 2], boxes[:, 3]
    bh = jnp.maximum(y1 - y0, 1.0)
    bw = jnp.maximum(x1 - x0, 1.0)
    k = (jnp.arange(RS, dtype=jnp.float32) + 0.5) / RS
    sy = y0[:, None] + k[None, :] * bh[:, None] - 0.5
    sx = x0[:, None] + k[None, :] * bw[:, None] - 0.5
    YY = sy[:, :, None]
    XX = sx[:, None, :]
    yf = jnp.floor(YY); xf = jnp.floor(XX)
    wy = YY - yf; wx = XX - xf
    y0i = jnp.clip(yf, 0, Hs - 1).astype(jnp.int32)
    y1i = jnp.clip(yf + 1, 0, Hs - 1).astype(jnp.int32)
    x0i = jnp.clip(xf, 0, Ws - 1).astype(jnp.int32)
    x1i = jnp.clip(xf + 1, 0, Ws - 1).astype(jnp.int32)
    g = lambda yi, xi: image[:, yi, xi]
    v = (g(y0i, x0i) * (1 - wy) * (1 - wx) + g(y0i, x1i) * (1 - wy) * wx
         + g(y1i, x0i) * wy * (1 - wx) + g(y1i, x1i) * wy * wx)
    return jnp.transpose(v, (1, 0, 2, 3))


def _roi_kernel(yidx_ref, imgp_ref, idx_ref, wx_ref, wyb_ref, out_ref):
    def body(n, _):
        slabs = [imgp_ref[yidx_ref[0, 0, n * RS + i]] for i in range(RS)]
        v = jnp.concatenate(slabs, axis=0)
        idx = idx_ref[n]
        idxm = jnp.broadcast_to(idx & 127, (2 * RS, 72))
        chunk = jnp.broadcast_to(idx >> 7, (2 * RS, 72))
        sel = jnp.zeros((2 * RS, 72), jnp.float32)
        for k in range(12):
            vk = v[:, k * 128:(k + 1) * 128]
            t = jnp.take_along_axis(vk, idxm, axis=1)
            sel = sel + jnp.where(chunk == k, t, 0.0)
        wx = jnp.broadcast_to(wx_ref[n], (2 * RS, 36))
        selx = sel[:, :36] * (1.0 - wx) + sel[:, 36:] * wx
        s2 = selx.reshape(RS, 2, 36)
        wy = wyb_ref[n]
        out_ref[n] = s2[:, 0, :] * (1.0 - wy) + s2[:, 1, :] * wy
        return ()
    jax.lax.fori_loop(0, yidx_ref.shape[2] // RS, body, ())


def _roi_extract(images_nhwc_flat, boxes):
    Bq = images_nhwc_flat.shape[0]
    N = boxes.shape[1]
    imgp = jnp.stack(
        [images_nhwc_flat,
         jnp.concatenate([images_nhwc_flat[:, 1:], images_nhwc_flat[:, -1:]], 1)],
        axis=2)
    imgp = imgp.reshape(Bq * 512, 2, 1536)

    y0, x0, y1, x1 = boxes[..., 0], boxes[..., 1], boxes[..., 2], boxes[..., 3]
    bh = jnp.maximum(y1 - y0, 1.0)
    bw = jnp.maximum(x1 - x0, 1.0)
    k = (jnp.arange(RS, dtype=jnp.float32) + 0.5) / RS
    sy = y0[..., None] + k * bh[..., None] - 0.5
    sx = x0[..., None] + k * bw[..., None] - 0.5
    yf = jnp.floor(sy)
    wy = jnp.where(yf < 0, 0.0, sy - yf)
    yidx = jnp.clip(yf, 0, 511).astype(jnp.int32)
    xf = jnp.floor(sx)
    wx = jnp.where(xf < 0, 0.0, sx - xf)
    x0i = jnp.clip(xf, 0, 511).astype(jnp.int32)
    x1i = jnp.clip(xf + 1, 0, 511).astype(jnp.int32)
    c3 = jnp.arange(3, dtype=jnp.int32)
    l0 = (x0i[..., None] * 3 + c3).reshape(Bq, N, 36)
    l1 = (x1i[..., None] * 3 + c3).reshape(Bq, N, 36)
    idx72 = jnp.concatenate([l0, l1], axis=-1).reshape(Bq * N, 1, 72)
    wx36 = jnp.repeat(wx, 3, axis=-1).reshape(Bq * N, 1, 36)
    wyb = jnp.broadcast_to(wy[..., None], (Bq, N, RS, 36)).reshape(Bq * N, RS, 36)

    out = pl.pallas_call(
        _roi_kernel,
        grid=(Bq,),
        in_specs=[
            pl.BlockSpec((1, 1, N * RS), lambda b: (b, 0, 0),
                         memory_space=pltpu.SMEM),
            pl.BlockSpec((512, 2, 1536), lambda b: (b, 0, 0)),
            pl.BlockSpec((N, 1, 72), lambda b: (b, 0, 0)),
            pl.BlockSpec((N, 1, 36), lambda b: (b, 0, 0)),
            pl.BlockSpec((N, RS, 36), lambda b: (b, 0, 0)),
        ],
        out_specs=pl.BlockSpec((N, RS, 36), lambda b: (b, 0, 0)),
        out_shape=jax.ShapeDtypeStruct((Bq * N, RS, 36), jnp.float32),
        compiler_params=pltpu.CompilerParams(
            dimension_semantics=("parallel",),
        ),
    )(yidx.reshape(Bq, 1, N * RS), imgp, idx72, wx36, wyb)
    return out


def _pool_proj_kernel(scores_ref, lat_ref, ow_ref, ob_ref, out_ref):
    s = scores_ref[0]
    lat = lat_ref[0]
    h = jnp.dot(s, lat, preferred_element_type=jnp.float32)
    out_ref[0] = (jnp.dot(h, ow_ref[...], preferred_element_type=jnp.float32)
                  + ob_ref[...])


def _pool_proj(scores, lat, ow, ob):
    Bq, N, D = lat.shape
    out = pl.pallas_call(
        _pool_proj_kernel,
        grid=(Bq,),
        in_specs=[
            pl.BlockSpec((1, 1, N), lambda b: (b, 0, 0)),
            pl.BlockSpec((1, N, D), lambda b: (b, 0, 0)),
            pl.BlockSpec((D, NC), lambda b: (0, 0)),
            pl.BlockSpec((1, NC), lambda b: (0, 0)),
        ],
        out_specs=pl.BlockSpec((1, 1, NC), lambda b: (b, 0, 0)),
        out_shape=jax.ShapeDtypeStruct((Bq, 1, NC), jnp.float32),
        compiler_params=pltpu.CompilerParams(
            dimension_semantics=("arbitrary",),
        ),
    )(scores.reshape(Bq, 1, N), lat, ow, ob.reshape(1, NC))
    return out.reshape(Bq, NC)


def kernel(images, gw1, gb1, gw2, gb2, gw3, gb3, gw4, gb4,
           rw1, rb1, rw2, rb2, ow, ob):
    Bq, _, Hi, Wi = images.shape
    g = _conv2d(_conv2d(_conv2d(_conv2d(images, gw1, gb1, 2), gw2, gb2, 2),
                        gw3, gb3, 2), gw4, gb4, 2)
    Hg, Wg = g.shape[2], g.shape[3]
    rr, cc = jnp.meshgrid(jnp.arange(Hg), jnp.arange(Wg), indexing='ij')
    cells = jnp.stack([rr, cc]).astype(images.dtype)[None]
    cr = jnp.array([Hi / Hg, Wi / Wg], images.dtype)[None, :, None, None]
    sr = jnp.array([Hi, Wi], images.dtype)[None, :, None, None]
    centers = cr * (jax.nn.sigmoid(g[:, 0:2]) + cells)
    sizes = sr * jax.nn.sigmoid(g[:, 2:4])
    bb = jnp.concatenate([centers - sizes / 2, centers + sizes / 2], axis=1)
    bb = jnp.stack([jnp.maximum(bb[:, 0], 0.0),
                    jnp.maximum(bb[:, 1], 0.0),
                    jnp.minimum(bb[:, 2], Hi - 1.0),
                    jnp.minimum(bb[:, 3], Wi - 1.0)], axis=1)
    boxes = jnp.floor(bb)
    boxes = boxes.transpose(0, 2, 3, 1).reshape(Bq, Hg * Wg, 4)
    scores = jax.nn.softmax(g[:, 4].reshape(Bq, -1), axis=-1)
    imgflat = images.transpose(0, 2, 3, 1).reshape(Bq, Hi, Wi * 3)
    regions = _roi_extract(imgflat, boxes)
    regions = regions.reshape(Bq * Hg * Wg, RS, RS, 3).transpose(0, 3, 1, 2)
    lat = _conv2d(_conv2d(regions, rw1, rb1, 2), rw2, rb2, 2)
    lat = lat.reshape(Bq, Hg * Wg, -1)
    logits = _pool_proj(scores, lat, ow, ob)
    return logits, scores.reshape(Bq, 1, Hg, Wg)


# speedup vs baseline: 2.2793x; 1.2331x over previous
import jax
import jax.numpy as jnp
import numpy as np
from jax.experimental import pallas as pl
from jax.experimental.pallas import tpu as pltpu

B, H, W = 16, 512, 512
RS = 12
NC = 80


def _conv2d(x, w, b, s):
    y = jax.lax.conv_general_dilated(x, w, (s, s), ((1, 1), (1, 1)),
                                     dimension_numbers=('NCHW', 'OIHW', 'NCHW'))
    return y + b[None, :, None, None]


def _extract_roi_one(image, boxes):
    Hs, Ws = image.shape[1], image.shape[2]
    y0, x0, y1, x1 = boxes[:, 0], boxes[:, 1], boxes[:, 2], boxes[:, 3]
    bh = jnp.maximum(y1 - y0, 1.0)
    bw = jnp.maximum(x1 - x0, 1.0)
    k = (jnp.arange(RS, dtype=jnp.float32) + 0.5) / RS
    sy = y0[:, None] + k[None, :] * bh[:, None] - 0.5
    sx = x0[:, None] + k[None, :] * bw[:, None] - 0.5
    YY = sy[:, :, None]
    XX = sx[:, None, :]
    yf = jnp.floor(YY); xf = jnp.floor(XX)
    wy = YY - yf; wx = XX - xf
    y0i = jnp.clip(yf, 0, Hs - 1).astype(jnp.int32)
    y1i = jnp.clip(yf + 1, 0, Hs - 1).astype(jnp.int32)
    x0i = jnp.clip(xf, 0, Ws - 1).astype(jnp.int32)
    x1i = jnp.clip(xf + 1, 0, Ws - 1).astype(jnp.int32)
    g = lambda yi, xi: image[:, yi, xi]
    v = (g(y0i, x0i) * (1 - wy) * (1 - wx) + g(y0i, x1i) * (1 - wy) * wx
         + g(y1i, x0i) * wy * (1 - wx) + g(y1i, x1i) * wy * wx)
    return jnp.transpose(v, (1, 0, 2, 3))


def _roi_kernel(yidx_ref, imgp_ref, idx_ref, wx_ref, wyb_ref, out_ref):
    G = 4

    def _tree_sum(parts):
        while len(parts) > 1:
            parts = [a + b for a, b in zip(parts[::2], parts[1::2])] + \
                (parts[-1:] if len(parts) % 2 else [])
        return parts[0]

    def _one_box(n):
        slabs = [imgp_ref[yidx_ref[0, 0, n * RS + i]] for i in range(RS)]
        lo = jnp.concatenate([s[0:1] for s in slabs], axis=0)
        hi = jnp.concatenate([s[1:2] for s in slabs], axis=0)
        idx = idx_ref[n]
        idxm = jnp.broadcast_to(idx & 127, (RS, 72))
        chunk = jnp.broadcast_to(idx >> 7, (RS, 72))
        plo, phi = [], []
        for k in range(12):
            m = chunk == k
            tlo = jnp.take_along_axis(lo[:, k * 128:(k + 1) * 128], idxm, axis=1)
            thi = jnp.take_along_axis(hi[:, k * 128:(k + 1) * 128], idxm, axis=1)
            plo.append(jnp.where(m, tlo, 0.0))
            phi.append(jnp.where(m, thi, 0.0))
        sel_lo = _tree_sum(plo)
        sel_hi = _tree_sum(phi)
        wx = jnp.broadcast_to(wx_ref[n], (RS, 36))
        sxl = sel_lo[:, :36] * (1.0 - wx) + sel_lo[:, 36:] * wx
        sxh = sel_hi[:, :36] * (1.0 - wx) + sel_hi[:, 36:] * wx
        wy = wyb_ref[n]
        out_ref[n] = sxl * (1.0 - wy) + sxh * wy

    def body(n0, _):
        for g in range(G):
            _one_box(n0 * G + g)
        return ()
    jax.lax.fori_loop(0, yidx_ref.shape[2] // (RS * G), body, ())


def _roi_extract(images_nhwc_flat, boxes):
    Bq = images_nhwc_flat.shape[0]
    N = boxes.shape[1]
    imgp = jnp.stack(
        [images_nhwc_flat,
         jnp.concatenate([images_nhwc_flat[:, 1:], images_nhwc_flat[:, -1:]], 1)],
        axis=2)
    imgp = imgp.reshape(Bq * 512, 2, 1536)

    y0, x0, y1, x1 = boxes[..., 0], boxes[..., 1], boxes[..., 2], boxes[..., 3]
    bh = jnp.maximum(y1 - y0, 1.0)
    bw = jnp.maximum(x1 - x0, 1.0)
    k = (jnp.arange(RS, dtype=jnp.float32) + 0.5) / RS
    sy = y0[..., None] + k * bh[..., None] - 0.5
    sx = x0[..., None] + k * bw[..., None] - 0.5
    yf = jnp.floor(sy)
    wy = jnp.where(yf < 0, 0.0, sy - yf)
    yidx = jnp.clip(yf, 0, 511).astype(jnp.int32)
    xf = jnp.floor(sx)
    wx = jnp.where(xf < 0, 0.0, sx - xf)
    x0i = jnp.clip(xf, 0, 511).astype(jnp.int32)
    x1i = jnp.clip(xf + 1, 0, 511).astype(jnp.int32)
    c3 = jnp.arange(3, dtype=jnp.int32)
    l0 = (x0i[..., None] * 3 + c3).reshape(Bq, N, 36)
    l1 = (x1i[..., None] * 3 + c3).reshape(Bq, N, 36)
    idx72 = jnp.concatenate([l0, l1], axis=-1).reshape(Bq * N, 1, 72)
    wx36 = jnp.repeat(wx, 3, axis=-1).reshape(Bq * N, 1, 36)
    wyb = jnp.broadcast_to(wy[..., None], (Bq, N, RS, 36)).reshape(Bq * N, RS, 36)

    out = pl.pallas_call(
        _roi_kernel,
        grid=(Bq,),
        in_specs=[
            pl.BlockSpec((1, 1, N * RS), lambda b: (b, 0, 0),
                         memory_space=pltpu.SMEM),
            pl.BlockSpec((512, 2, 1536), lambda b: (b, 0, 0)),
            pl.BlockSpec((N, 1, 72), lambda b: (b, 0, 0)),
            pl.BlockSpec((N, 1, 36), lambda b: (b, 0, 0)),
            pl.BlockSpec((N, RS, 36), lambda b: (b, 0, 0)),
        ],
        out_specs=pl.BlockSpec((N, RS, 36), lambda b: (b, 0, 0)),
        out_shape=jax.ShapeDtypeStruct((Bq * N, RS, 36), jnp.float32),
        compiler_params=pltpu.CompilerParams(
            dimension_semantics=("parallel",),
        ),
    )(yidx.reshape(Bq, 1, N * RS), imgp, idx72, wx36, wyb)
    return out


def _pool_proj_kernel(scores_ref, lat_ref, ow_ref, ob_ref, out_ref):
    s = scores_ref[0]
    lat = lat_ref[0]
    h = jnp.dot(s, lat, preferred_element_type=jnp.float32)
    out_ref[0] = (jnp.dot(h, ow_ref[...], preferred_element_type=jnp.float32)
                  + ob_ref[...])


def _pool_proj(scores, lat, ow, ob):
    Bq, N, D = lat.shape
    out = pl.pallas_call(
        _pool_proj_kernel,
        grid=(Bq,),
        in_specs=[
            pl.BlockSpec((1, 1, N), lambda b: (b, 0, 0)),
            pl.BlockSpec((1, N, D), lambda b: (b, 0, 0)),
            pl.BlockSpec((D, NC), lambda b: (0, 0)),
            pl.BlockSpec((1, NC), lambda b: (0, 0)),
        ],
        out_specs=pl.BlockSpec((1, 1, NC), lambda b: (b, 0, 0)),
        out_shape=jax.ShapeDtypeStruct((Bq, 1, NC), jnp.float32),
        compiler_params=pltpu.CompilerParams(
            dimension_semantics=("arbitrary",),
        ),
    )(scores.reshape(Bq, 1, N), lat, ow, ob.reshape(1, NC))
    return out.reshape(Bq, NC)


def kernel(images, gw1, gb1, gw2, gb2, gw3, gb3, gw4, gb4,
           rw1, rb1, rw2, rb2, ow, ob):
    Bq, _, Hi, Wi = images.shape
    g = _conv2d(_conv2d(_conv2d(_conv2d(images, gw1, gb1, 2), gw2, gb2, 2),
                        gw3, gb3, 2), gw4, gb4, 2)
    Hg, Wg = g.shape[2], g.shape[3]
    rr, cc = jnp.meshgrid(jnp.arange(Hg), jnp.arange(Wg), indexing='ij')
    cells = jnp.stack([rr, cc]).astype(images.dtype)[None]
    cr = jnp.array([Hi / Hg, Wi / Wg], images.dtype)[None, :, None, None]
    sr = jnp.array([Hi, Wi], images.dtype)[None, :, None, None]
    centers = cr * (jax.nn.sigmoid(g[:, 0:2]) + cells)
    sizes = sr * jax.nn.sigmoid(g[:, 2:4])
    bb = jnp.concatenate([centers - sizes / 2, centers + sizes / 2], axis=1)
    bb = jnp.stack([jnp.maximum(bb[:, 0], 0.0),
                    jnp.maximum(bb[:, 1], 0.0),
                    jnp.minimum(bb[:, 2], Hi - 1.0),
                    jnp.minimum(bb[:, 3], Wi - 1.0)], axis=1)
    boxes = jnp.floor(bb)
    boxes = boxes.transpose(0, 2, 3, 1).reshape(Bq, Hg * Wg, 4)
    scores = jax.nn.softmax(g[:, 4].reshape(Bq, -1), axis=-1)
    imgflat = images.transpose(0, 2, 3, 1).reshape(Bq, Hi, Wi * 3)
    regions = _roi_extract(imgflat, boxes)
    regions = regions.reshape(Bq * Hg * Wg, RS, RS, 3).transpose(0, 3, 1, 2)
    lat = _conv2d(_conv2d(regions, rw1, rb1, 2), rw2, rb2, 2)
    lat = lat.reshape(Bq, Hg * Wg, -1)
    logits = _pool_proj(scores, lat, ow, ob)
    return logits, scores.reshape(Bq, 1, Hg, Wg)
